# baseline (speedup 1.0000x reference)
import jax
import jax.numpy as jnp
from jax import lax
from jax.experimental import pallas as pl
from jax.experimental.pallas import tpu as pltpu
from jax.experimental.pallas import tpu_sc as plsc

NC = 2
NS = 16
L = 16
NW = NC * NS

B = 16384
K = 32
BPW = B // NW
GRP = 16
NSLOT = 8


def _body(ut_hbm, it_hbm, uids_hbm, iids_hbm, out_hbm,
          uidsv, iidsv, uwins, iwins, outv, sem):
  c = lax.axis_index("c")
  s = lax.axis_index("s")
  w = s * NC + c

  pltpu.sync_copy(uids_hbm.at[w], uidsv)
  pltpu.sync_copy(iids_hbm.at[w], iidsv)

  lanes = lax.iota(jnp.int32, L)

  def group(g, carry):
    u16 = uidsv[pl.ds(g * GRP, GRP)]
    i16 = iidsv[pl.ds(g * GRP, GRP)]
    acc = jnp.zeros((L,), jnp.float32)
    for h in range(GRP // NSLOT):
      copies = []
      for j in range(NSLOT):
        jj = h * NSLOT + j
        ublk = (u16[jj] // 128) * 128
        iblk = (i16[jj] // 128) * 128
        copies.append(pltpu.async_copy(
            ut_hbm.at[:, pl.ds(ublk, 128)], uwins.at[j], sem))
        copies.append(pltpu.async_copy(
            it_hbm.at[:, pl.ds(iblk, 128)], iwins.at[j], sem))
      for cp in copies:
        cp.wait()
      for j in range(NSLOT):
        jj = h * NSLOT + j
        ulane = jnp.full((L,), u16[jj] % 128, jnp.int32)
        ilane = jnp.full((L,), i16[jj] % 128, jnp.int32)
        jfull = jnp.full((L,), j, jnp.int32)
        u0 = plsc.load_gather(uwins, [jfull, lanes, ulane])
        u1 = plsc.load_gather(uwins, [jfull, lanes + L, ulane])
        v0 = plsc.load_gather(iwins, [jfull, lanes, ilane])
        v1 = plsc.load_gather(iwins, [jfull, lanes + L, ilane])
        part = u0 * v0 + u1 * v1
        tot = lax.reduce_sum_p.bind(part, axes=(0,))
        acc = jnp.where(lanes == jj, tot, acc)
    outv[pl.ds(g * GRP, GRP)] = acc
    return carry

  lax.fori_loop(0, BPW // GRP, group, 0)

  pltpu.sync_copy(outv, out_hbm.at[pl.ds(w * BPW, BPW)])


@jax.jit
def _mf(ut, it, uids, iids):
  mesh = plsc.VectorSubcoreMesh(core_axis_name="c", subcore_axis_name="s",
                                num_cores=NC, num_subcores=NS)
  f = pl.kernel(
      _body,
      out_type=jax.ShapeDtypeStruct((B,), jnp.float32),
      mesh=mesh,
      scratch_types=[
          pltpu.VMEM((BPW,), jnp.int32),
          pltpu.VMEM((BPW,), jnp.int32),
          pltpu.VMEM((NSLOT, K, 128), jnp.float32),
          pltpu.VMEM((NSLOT, K, 128), jnp.float32),
          pltpu.VMEM((BPW,), jnp.float32),
          pltpu.SemaphoreType.DMA,
      ],
      compiler_params=pltpu.CompilerParams(needs_layout_passes=False,
                                           use_tc_tiling_on_sc=True),
  )
  return f(ut, it, uids, iids)


def kernel(x, user_m, item_m):
  ut = user_m.T
  it = item_m.T
  uids = x[:, 0].astype(jnp.int32).reshape(NW, BPW)
  iids = x[:, 1].astype(jnp.int32).reshape(NW, BPW)
  return _mf(ut, it, uids, iids)


# speedup vs baseline: 1.0221x; 1.0221x over previous
import jax
import jax.numpy as jnp
from jax import lax
from jax.experimental import pallas as pl
from jax.experimental.pallas import tpu as pltpu
from jax.experimental.pallas import tpu_sc as plsc

NC = 2
NS = 16
L = 16
NW = NC * NS

B = 16384
K = 32
BPW = B // NW
GRP = 16
NB = 4
NGRP = BPW // GRP


def _body(ut_hbm, it_hbm, uids_hbm, iids_hbm, out_hbm,
          uidsv, iidsv, uwins, iwins, outv, sem0, sem1):
  c = lax.axis_index("c")
  s = lax.axis_index("s")
  w = s * NC + c

  pltpu.sync_copy(uids_hbm.at[w], uidsv)
  pltpu.sync_copy(iids_hbm.at[w], iidsv)

  lanes = lax.iota(jnp.int32, L)
  sems = (sem0, sem1)

  def fire(u16, i16, t, p):
    for j in range(NB):
      jj = t * NB + j
      ublk = (u16[jj] // 128) * 128
      iblk = (i16[jj] // 128) * 128
      pltpu.async_copy(ut_hbm.at[:, pl.ds(ublk, 128)],
                       uwins.at[p, j], sems[p])
      pltpu.async_copy(it_hbm.at[:, pl.ds(iblk, 128)],
                       iwins.at[p, j], sems[p])

  def drain(p):
    for j in range(NB):
      pltpu.make_async_copy(ut_hbm.at[:, pl.ds(0, 128)],
                            uwins.at[p, j], sems[p]).wait()
      pltpu.make_async_copy(it_hbm.at[:, pl.ds(0, 128)],
                            iwins.at[p, j], sems[p]).wait()

  def ids_at(g):
    return (uidsv[pl.ds(g * GRP, GRP)], iidsv[pl.ds(g * GRP, GRP)])

  u0_16, i0_16 = ids_at(0)
  fire(u0_16, i0_16, 0, 0)

  def group(g, carry):
    u16, i16 = ids_at(g)
    gn = jnp.minimum(g + 1, NGRP - 1)
    un16, in16 = ids_at(gn)
    acc = jnp.zeros((L,), jnp.float32)
    for t in range(GRP // NB):
      p = t % 2
      if t < GRP // NB - 1:
        fire(u16, i16, t + 1, 1 - p)
      else:
        fire(un16, in16, 0, 1 - p)
      drain(p)
      for j in range(NB):
        jj = t * NB + j
        ulane = jnp.full((L,), u16[jj] % 128, jnp.int32)
        ilane = jnp.full((L,), i16[jj] % 128, jnp.int32)
        pfull = jnp.full((L,), p, jnp.int32)
        jfull = jnp.full((L,), j, jnp.int32)
        u0 = plsc.load_gather(uwins, [pfull, jfull, lanes, ulane])
        u1 = plsc.load_gather(uwins, [pfull, jfull, lanes + L, ulane])
        v0 = plsc.load_gather(iwins, [pfull, jfull, lanes, ilane])
        v1 = plsc.load_gather(iwins, [pfull, jfull, lanes + L, ilane])
        part = u0 * v0 + u1 * v1
        tot = lax.reduce_sum_p.bind(part, axes=(0,))
        acc = jnp.where(lanes == jj, tot, acc)
    outv[pl.ds(g * GRP, GRP)] = acc
    return carry

  lax.fori_loop(0, NGRP, group, 0)

  drain(0)

  pltpu.sync_copy(outv, out_hbm.at[pl.ds(w * BPW, BPW)])


@jax.jit
def _mf(ut, it, uids, iids):
  mesh = plsc.VectorSubcoreMesh(core_axis_name="c", subcore_axis_name="s",
                                num_cores=NC, num_subcores=NS)
  f = pl.kernel(
      _body,
      out_type=jax.ShapeDtypeStruct((B,), jnp.float32),
      mesh=mesh,
      scratch_types=[
          pltpu.VMEM((BPW,), jnp.int32),
          pltpu.VMEM((BPW,), jnp.int32),
          pltpu.VMEM((2, NB, K, 128), jnp.float32),
          pltpu.VMEM((2, NB, K, 128), jnp.float32),
          pltpu.VMEM((BPW,), jnp.float32),
          pltpu.SemaphoreType.DMA,
          pltpu.SemaphoreType.DMA,
      ],
      compiler_params=pltpu.CompilerParams(needs_layout_passes=False,
                                           use_tc_tiling_on_sc=True),
  )
  return f(ut, it, uids, iids)


def kernel(x, user_m, item_m):
  ut = user_m.T
  it = item_m.T
  uids = x[:, 0].astype(jnp.int32).reshape(NW, BPW)
  iids = x[:, 1].astype(jnp.int32).reshape(NW, BPW)
  return _mf(ut, it, uids, iids)
